# baseline (speedup 1.0000x reference)
import functools

import jax
import jax.numpy as jnp
from jax import lax
from jax.experimental import pallas as pl
from jax.experimental.pallas import tpu as pltpu
from jax.experimental.pallas import tpu_sc as plsc

_B, _C, _H, _W = 1024, 256, 14, 14
_HW = _H * _W
_SR = _C * _HW // 128
_NW = 32
_EPW = _B // _NW
_RPW = _EPW * _SR
_KC = 64
_DC = 64
_MC = 64
_NCH = (_SR + 15) // 16
_MXN = 255 * _EPW + _MC


def _sc_body(x_hbm, idx_hbm, m_hbm, z_hbm, out_hbm,
             idx_v, gidx_v, horiz_v, mrow_v, kept_v, drop_v, mix_v,
             mgid_v, mpos_v, mside_v,
             idxstage, dstage, mixidx,
             buf0, buf1, mbuf0, mbuf1, zbuf,
             msem, zsem, gsem0, gsem1, ssem0, ssem1,
             mgsem0, mgsem1, mssem0, mssem1):
    cid = lax.axis_index("c")
    sid = lax.axis_index("s")
    wid = sid * 2 + cid
    b0 = wid * _EPW

    pltpu.sync_copy(idx_hbm.at[pl.ds(b0, _EPW)], idx_v)
    for t in range(_EPW // 16):
        v = idx_v[pl.ds(16 * t, 16)]
        gidx_v[pl.ds(16 * t, 16)] = lax.shift_right_logical(v, 1)
        horiz_v[pl.ds(16 * t, 16)] = jnp.bitwise_and(v, 1) * 64
    pltpu.async_copy(m_hbm.at[gidx_v], mrow_v, msem).wait()
    pltpu.sync_copy(z_hbm, zbuf)

    iota = lax.iota(jnp.int32, 16)

    def comp(i, carry):
        nk, nd, nm, lastk, lastd, lastm = carry
        e = i // _NCH
        ch = i % _NCH
        k = ch * 16 + iota
        valid = k < _SR
        start = 128 * k
        c0 = start // _HW
        c1 = (start + 127) // _HW
        se = jnp.broadcast_to(e, (16,))
        he = plsc.load_gather(horiz_v, [se])
        w0 = plsc.load_gather(mrow_v, [se, he + lax.shift_right_logical(c0, 2)])
        b0v = jnp.bitwise_and(
            lax.shift_right_logical(w0, jnp.bitwise_and(c0, 3) * 8), 0xFF)
        k0 = b0v != 0
        w1 = plsc.load_gather(mrow_v, [se, he + lax.shift_right_logical(c1, 2)])
        b1v = jnp.bitwise_and(
            lax.shift_right_logical(w1, jnp.bitwise_and(c1, 3) * 8), 0xFF)
        k1 = b1v != 0
        gid = (b0 + e) * _SR + k
        uni = k0 == k1
        keepm = valid & uni & k0
        dropm = valid & uni & jnp.logical_not(k0)
        mixm = valid & jnp.logical_not(uni)
        pos = _HW * c1 - start
        mixword = gid | (pos << 20) | (k0.astype(jnp.int32) << 27)

        mi = keepm.astype(jnp.int32)
        plsc.store_scatter(kept_v, [nk + plsc.cumsum(mi) - 1], gid, mask=keepm)
        ck = jnp.sum(mi)
        mi = dropm.astype(jnp.int32)
        plsc.store_scatter(drop_v, [nd + plsc.cumsum(mi) - 1], gid, mask=dropm)
        cd = jnp.sum(mi)
        mi = mixm.astype(jnp.int32)
        plsc.store_scatter(mix_v, [nm + plsc.cumsum(mi) - 1], mixword,
                           mask=mixm)
        cm = jnp.sum(mi)
        lastk = jnp.maximum(lastk, jnp.max(jnp.where(keepm, gid, -1)))
        lastd = jnp.maximum(lastd, jnp.max(jnp.where(dropm, gid, -1)))
        lastm = jnp.maximum(lastm, jnp.max(jnp.where(mixm, mixword, -1)))
        return nk + ck, nd + cd, nm + cm, lastk, lastd, lastm

    zero = jnp.array(0, jnp.int32)
    neg = jnp.array(-1, jnp.int32)
    nk, nd, nm, lastk, lastd, lastm = lax.fori_loop(
        0, _EPW * _NCH, comp, (zero, zero, zero, neg, neg, neg))

    for t in range(_KC // 16):
        kept_v[pl.ds(nk + 16 * t, 16)] = jnp.broadcast_to(lastk, (16,))
    for t in range(_DC // 16):
        drop_v[pl.ds(nd + 16 * t, 16)] = jnp.broadcast_to(lastd, (16,))
    for t in range(_MC // 16):
        mix_v[pl.ds(nm + 16 * t, 16)] = jnp.broadcast_to(lastm, (16,))

    trips_d = (nd + _DC - 1) // _DC

    def dbody(g, _):
        for j in range(4):
            i = g * 4 + j

            @pl.when(i < trips_d)
            def _(i=i, j=j):
                for t in range(_DC // 16):
                    dstage[j, pl.ds(t * 16, 16)] = \
                        drop_v[pl.ds(i * _DC + t * 16, 16)]
                pltpu.make_async_copy(zbuf, out_hbm.at[dstage.at[j]],
                                      zsem).start()
        for j in range(4):

            @pl.when(g * 4 + j < trips_d)
            def _(j=j):
                pltpu.make_async_copy(zbuf, out_hbm.at[dstage.at[j]],
                                      zsem).wait()
        return 0

    lax.fori_loop(0, (trips_d + 3) // 4, dbody, 0)

    trips_k = (nk + _KC - 1) // _KC

    def stage_kept(i, p):
        for t in range(_KC // 16):
            idxstage[p, pl.ds(t * 16, 16)] = \
                kept_v[pl.ds(i * _KC + t * 16, 16)]

    def g_start(i, buf, sem):
        pltpu.make_async_copy(x_hbm.at[kept_v.at[pl.ds(i * _KC, _KC)]],
                              buf, sem).start()

    def g_wait(i, buf, sem):
        pltpu.make_async_copy(x_hbm.at[kept_v.at[pl.ds(i * _KC, _KC)]],
                              buf, sem).wait()

    def s_start(p, buf, sem):
        pltpu.make_async_copy(buf, out_hbm.at[idxstage.at[p]], sem).start()

    def s_wait(p, buf, sem):
        pltpu.make_async_copy(buf, out_hbm.at[idxstage.at[p]], sem).wait()

    stage_kept(0, 0)
    g_start(0, buf0, gsem0)

    def kbody(i, _):
        def half(p, pn, bufp, bufn, gsemp, gsemn, ssemp, ssemn):
            @pl.when(i + 1 < trips_k)
            def _():
                @pl.when(i >= 1)
                def _():
                    s_wait(pn, bufn, ssemn)
                stage_kept(i + 1, pn)
                g_start(i + 1, bufn, gsemn)

            g_wait(i, bufp, gsemp)
            s_start(p, bufp, ssemp)

        @pl.when(i % 2 == 0)
        def _():
            half(0, 1, buf0, buf1, gsem0, gsem1, ssem0, ssem1)

        @pl.when(i % 2 == 1)
        def _():
            half(1, 0, buf1, buf0, gsem1, gsem0, ssem1, ssem0)

        return 0

    lax.fori_loop(0, trips_k, kbody, 0)

    @pl.when(trips_k >= 2)
    def _():
        @pl.when(trips_k % 2 == 0)
        def _():
            s_wait(0, buf0, ssem0)

        @pl.when(trips_k % 2 == 1)
        def _():
            s_wait(1, buf1, ssem1)

    @pl.when(trips_k % 2 == 1)
    def _():
        s_wait(0, buf0, ssem0)

    @pl.when((trips_k >= 1) & (trips_k % 2 == 0))
    def _():
        s_wait(1, buf1, ssem1)

    trips_m = (nm + _MC - 1) // _MC

    def decbody(t, _):
        w = mix_v[pl.ds(t * 16, 16)]
        mgid_v[pl.ds(t * 16, 16)] = jnp.bitwise_and(w, 0xFFFFF)
        mpos_v[pl.ds(t * 16, 16)] = jnp.bitwise_and(
            lax.shift_right_logical(w, 20), 0x7F)
        mside_v[pl.ds(t * 16, 16)] = jnp.bitwise_and(
            lax.shift_right_logical(w, 27), 1)
        return 0

    lax.fori_loop(0, trips_m * (_MC // 16), decbody, 0)

    def stage_mix(i, p):
        for t in range(_MC // 16):
            mixidx[p, pl.ds(t * 16, 16)] = \
                mgid_v[pl.ds(i * _MC + t * 16, 16)]

    def mg_start(i, buf, sem):
        pltpu.make_async_copy(x_hbm.at[mgid_v.at[pl.ds(i * _MC, _MC)]],
                              buf, sem).start()

    def mg_wait(i, buf, sem):
        pltpu.make_async_copy(x_hbm.at[mgid_v.at[pl.ds(i * _MC, _MC)]],
                              buf, sem).wait()

    def ms_start(p, buf, sem):
        pltpu.make_async_copy(buf, out_hbm.at[mixidx.at[p]], sem).start()

    def ms_wait(p, buf, sem):
        pltpu.make_async_copy(buf, out_hbm.at[mixidx.at[p]], sem).wait()

    def patch(i, buf):
        def pbody(r, _):
            sidx = jnp.broadcast_to(i * _MC + r, (16,))
            pr = plsc.load_gather(mpos_v, [sidx])
            sd = plsc.load_gather(mside_v, [sidx])
            for j in range(8):
                lanes = 16 * j + iota
                left = (lanes < pr).astype(jnp.int32)
                keep = left == sd
                v = buf[r, pl.ds(16 * j, 16)]
                buf[r, pl.ds(16 * j, 16)] = jnp.where(keep, v, 0.0)
            return 0

        lax.fori_loop(0, _MC, pbody, 0)

    @pl.when(trips_m >= 1)
    def _():
        stage_mix(0, 0)
        mg_start(0, mbuf0, mgsem0)

    def mbody(i, _):
        def half(p, pn, bufp, bufn, gsemp, gsemn, ssemp, ssemn):
            @pl.when(i + 1 < trips_m)
            def _():
                @pl.when(i >= 1)
                def _():
                    ms_wait(pn, bufn, ssemn)
                stage_mix(i + 1, pn)
                mg_start(i + 1, bufn, gsemn)

            mg_wait(i, bufp, gsemp)
            patch(i, bufp)
            ms_start(p, bufp, ssemp)

        @pl.when(i % 2 == 0)
        def _():
            half(0, 1, mbuf0, mbuf1, mgsem0, mgsem1, mssem0, mssem1)

        @pl.when(i % 2 == 1)
        def _():
            half(1, 0, mbuf1, mbuf0, mgsem1, mgsem0, mssem1, mssem0)

        return 0

    lax.fori_loop(0, trips_m, mbody, 0)

    @pl.when(trips_m >= 2)
    def _():
        @pl.when(trips_m % 2 == 0)
        def _():
            ms_wait(0, mbuf0, mssem0)

        @pl.when(trips_m % 2 == 1)
        def _():
            ms_wait(1, mbuf1, mssem1)

    @pl.when(trips_m % 2 == 1)
    def _():
        ms_wait(0, mbuf0, mssem0)

    @pl.when((trips_m >= 1) & (trips_m % 2 == 0))
    def _():
        ms_wait(1, mbuf1, mssem1)


def kernel(X, idx, masks):
    n = masks.shape[0]
    xr = X.reshape(_B * _SR, 128)
    mi32 = lax.bitcast_convert_type(
        masks.view(jnp.uint8).reshape(n // 2, 128, 4), jnp.int32)
    z = jnp.zeros((_DC, 128), jnp.float32)

    mesh = plsc.VectorSubcoreMesh(core_axis_name="c", subcore_axis_name="s")
    run = functools.partial(
        pl.kernel, mesh=mesh,
        out_type=jax.ShapeDtypeStruct((_B * _SR, 128), jnp.float32),
        scratch_types=[
            pltpu.VMEM((_EPW,), jnp.int32),
            pltpu.VMEM((_EPW,), jnp.int32),
            pltpu.VMEM((_EPW,), jnp.int32),
            pltpu.VMEM((_EPW, 128), jnp.int32),
            pltpu.VMEM((_RPW + _KC,), jnp.int32),
            pltpu.VMEM((_RPW + _DC,), jnp.int32),
            pltpu.VMEM((_MXN,), jnp.int32),
            pltpu.VMEM((_MXN,), jnp.int32),
            pltpu.VMEM((_MXN,), jnp.int32),
            pltpu.VMEM((_MXN,), jnp.int32),
            pltpu.VMEM((2, _KC), jnp.int32),
            pltpu.VMEM((4, _DC), jnp.int32),
            pltpu.VMEM((2, _MC), jnp.int32),
            pltpu.VMEM((_KC, 128), jnp.float32),
            pltpu.VMEM((_KC, 128), jnp.float32),
            pltpu.VMEM((_MC, 128), jnp.float32),
            pltpu.VMEM((_MC, 128), jnp.float32),
            pltpu.VMEM((_DC, 128), jnp.float32),
            pltpu.SemaphoreType.DMA,
            pltpu.SemaphoreType.DMA,
            pltpu.SemaphoreType.DMA,
            pltpu.SemaphoreType.DMA,
            pltpu.SemaphoreType.DMA,
            pltpu.SemaphoreType.DMA,
            pltpu.SemaphoreType.DMA,
            pltpu.SemaphoreType.DMA,
            pltpu.SemaphoreType.DMA,
            pltpu.SemaphoreType.DMA,
        ],
        compiler_params=pltpu.CompilerParams(needs_layout_passes=False),
    )(_sc_body)
    out = run(xr, idx.astype(jnp.int32), mi32, z)
    return out.reshape(_B, _C, _H, _W)


# speedup vs baseline: 2.1655x; 2.1655x over previous
import functools

import jax
import jax.numpy as jnp
from jax import lax
from jax.experimental import pallas as pl
from jax.experimental.pallas import tpu as pltpu
from jax.experimental.pallas import tpu_sc as plsc

_B, _C, _H, _W = 1024, 256, 14, 14
_HW = _H * _W
_SR = _C * _HW // 128
_NW = 32
_EPW = _B // _NW
_RPW = _EPW * _SR
_KC = 64
_DC = 64
_MC = 64
_NCH = (_SR + 15) // 16
_MXN = 255 * _EPW + _MC


def _sc_body(x_hbm, bm_hbm, z_hbm, out_hbm,
             bmask_v, kept_v, drop_v, mix_v,
             mgid_v, mpos_v, mside_v,
             idxstage, dstage, mixidx,
             buf0, buf1, mbuf0, mbuf1, zbuf,
             zsem, gsem0, gsem1, ssem0, ssem1,
             mgsem0, mgsem1, mssem0, mssem1):
    cid = lax.axis_index("c")
    sid = lax.axis_index("s")
    wid = sid * 2 + cid
    b0 = wid * _EPW

    pltpu.sync_copy(bm_hbm.at[pl.ds(b0, _EPW)], bmask_v)
    pltpu.sync_copy(z_hbm, zbuf)

    iota = lax.iota(jnp.int32, 16)

    def comp(i, carry):
        nk, nd, nm, lastk, lastd, lastm = carry
        e = i // _NCH
        ch = i % _NCH
        k = ch * 16 + iota
        valid = k < _SR
        start = 128 * k
        c0 = start // _HW
        c1 = (start + 127) // _HW
        se = jnp.broadcast_to(e, (16,))
        k0 = plsc.load_gather(bmask_v, [se, c0]) != 0.0
        k1 = plsc.load_gather(bmask_v, [se, c1]) != 0.0
        gid = (b0 + e) * _SR + k
        uni = k0 == k1
        keepm = valid & uni & k0
        dropm = valid & uni & jnp.logical_not(k0)
        mixm = valid & jnp.logical_not(uni)
        pos = _HW * c1 - start
        mixword = gid | (pos << 20) | (k0.astype(jnp.int32) << 27)

        mi = keepm.astype(jnp.int32)
        plsc.store_scatter(kept_v, [nk + plsc.cumsum(mi) - 1], gid, mask=keepm)
        ck = jnp.sum(mi)
        mi = dropm.astype(jnp.int32)
        plsc.store_scatter(drop_v, [nd + plsc.cumsum(mi) - 1], gid, mask=dropm)
        cd = jnp.sum(mi)
        mi = mixm.astype(jnp.int32)
        plsc.store_scatter(mix_v, [nm + plsc.cumsum(mi) - 1], mixword,
                           mask=mixm)
        cm = jnp.sum(mi)
        lastk = jnp.maximum(lastk, jnp.max(jnp.where(keepm, gid, -1)))
        lastd = jnp.maximum(lastd, jnp.max(jnp.where(dropm, gid, -1)))
        lastm = jnp.maximum(lastm, jnp.max(jnp.where(mixm, mixword, -1)))
        return nk + ck, nd + cd, nm + cm, lastk, lastd, lastm

    zero = jnp.array(0, jnp.int32)
    neg = jnp.array(-1, jnp.int32)
    nk, nd, nm, lastk, lastd, lastm = lax.fori_loop(
        0, _EPW * _NCH, comp, (zero, zero, zero, neg, neg, neg))

    for t in range(_KC // 16):
        kept_v[pl.ds(nk + 16 * t, 16)] = jnp.broadcast_to(lastk, (16,))
    for t in range(_DC // 16):
        drop_v[pl.ds(nd + 16 * t, 16)] = jnp.broadcast_to(lastd, (16,))
    for t in range(_MC // 16):
        mix_v[pl.ds(nm + 16 * t, 16)] = jnp.broadcast_to(lastm, (16,))

    trips_d = (nd + _DC - 1) // _DC

    def dbody(g, _):
        for j in range(4):
            i = g * 4 + j

            @pl.when(i < trips_d)
            def _(i=i, j=j):
                for t in range(_DC // 16):
                    dstage[j, pl.ds(t * 16, 16)] = \
                        drop_v[pl.ds(i * _DC + t * 16, 16)]
                pltpu.make_async_copy(zbuf, out_hbm.at[dstage.at[j]],
                                      zsem).start()
        for j in range(4):

            @pl.when(g * 4 + j < trips_d)
            def _(j=j):
                pltpu.make_async_copy(zbuf, out_hbm.at[dstage.at[j]],
                                      zsem).wait()
        return 0

    lax.fori_loop(0, (trips_d + 3) // 4, dbody, 0)

    trips_k = (nk + _KC - 1) // _KC

    def stage_kept(i, p):
        for t in range(_KC // 16):
            idxstage[p, pl.ds(t * 16, 16)] = \
                kept_v[pl.ds(i * _KC + t * 16, 16)]

    def g_start(i, buf, sem):
        pltpu.make_async_copy(x_hbm.at[kept_v.at[pl.ds(i * _KC, _KC)]],
                              buf, sem).start()

    def g_wait(i, buf, sem):
        pltpu.make_async_copy(x_hbm.at[kept_v.at[pl.ds(i * _KC, _KC)]],
                              buf, sem).wait()

    def s_start(p, buf, sem):
        pltpu.make_async_copy(buf, out_hbm.at[idxstage.at[p]], sem).start()

    def s_wait(p, buf, sem):
        pltpu.make_async_copy(buf, out_hbm.at[idxstage.at[p]], sem).wait()

    stage_kept(0, 0)
    g_start(0, buf0, gsem0)

    def kbody(i, _):
        def half(p, pn, bufp, bufn, gsemp, gsemn, ssemp, ssemn):
            @pl.when(i + 1 < trips_k)
            def _():
                @pl.when(i >= 1)
                def _():
                    s_wait(pn, bufn, ssemn)
                stage_kept(i + 1, pn)
                g_start(i + 1, bufn, gsemn)

            g_wait(i, bufp, gsemp)
            s_start(p, bufp, ssemp)

        @pl.when(i % 2 == 0)
        def _():
            half(0, 1, buf0, buf1, gsem0, gsem1, ssem0, ssem1)

        @pl.when(i % 2 == 1)
        def _():
            half(1, 0, buf1, buf0, gsem1, gsem0, ssem1, ssem0)

        return 0

    lax.fori_loop(0, trips_k, kbody, 0)

    @pl.when(trips_k >= 2)
    def _():
        @pl.when(trips_k % 2 == 0)
        def _():
            s_wait(0, buf0, ssem0)

        @pl.when(trips_k % 2 == 1)
        def _():
            s_wait(1, buf1, ssem1)

    @pl.when(trips_k % 2 == 1)
    def _():
        s_wait(0, buf0, ssem0)

    @pl.when((trips_k >= 1) & (trips_k % 2 == 0))
    def _():
        s_wait(1, buf1, ssem1)

    trips_m = (nm + _MC - 1) // _MC

    def decbody(t, _):
        w = mix_v[pl.ds(t * 16, 16)]
        mgid_v[pl.ds(t * 16, 16)] = jnp.bitwise_and(w, 0xFFFFF)
        mpos_v[pl.ds(t * 16, 16)] = jnp.bitwise_and(
            lax.shift_right_logical(w, 20), 0x7F)
        mside_v[pl.ds(t * 16, 16)] = jnp.bitwise_and(
            lax.shift_right_logical(w, 27), 1)
        return 0

    lax.fori_loop(0, trips_m * (_MC // 16), decbody, 0)

    def stage_mix(i, p):
        for t in range(_MC // 16):
            mixidx[p, pl.ds(t * 16, 16)] = \
                mgid_v[pl.ds(i * _MC + t * 16, 16)]

    def mg_start(i, buf, sem):
        pltpu.make_async_copy(x_hbm.at[mgid_v.at[pl.ds(i * _MC, _MC)]],
                              buf, sem).start()

    def mg_wait(i, buf, sem):
        pltpu.make_async_copy(x_hbm.at[mgid_v.at[pl.ds(i * _MC, _MC)]],
                              buf, sem).wait()

    def ms_start(p, buf, sem):
        pltpu.make_async_copy(buf, out_hbm.at[mixidx.at[p]], sem).start()

    def ms_wait(p, buf, sem):
        pltpu.make_async_copy(buf, out_hbm.at[mixidx.at[p]], sem).wait()

    def patch(i, buf):
        def pbody(r, _):
            sidx = jnp.broadcast_to(i * _MC + r, (16,))
            pr = plsc.load_gather(mpos_v, [sidx])
            sd = plsc.load_gather(mside_v, [sidx])
            for j in range(8):
                lanes = 16 * j + iota
                left = (lanes < pr).astype(jnp.int32)
                keep = left == sd
                v = buf[r, pl.ds(16 * j, 16)]
                buf[r, pl.ds(16 * j, 16)] = jnp.where(keep, v, 0.0)
            return 0

        lax.fori_loop(0, _MC, pbody, 0)

    @pl.when(trips_m >= 1)
    def _():
        stage_mix(0, 0)
        mg_start(0, mbuf0, mgsem0)

    def mbody(i, _):
        def half(p, pn, bufp, bufn, gsemp, gsemn, ssemp, ssemn):
            @pl.when(i + 1 < trips_m)
            def _():
                @pl.when(i >= 1)
                def _():
                    ms_wait(pn, bufn, ssemn)
                stage_mix(i + 1, pn)
                mg_start(i + 1, bufn, gsemn)

            mg_wait(i, bufp, gsemp)
            patch(i, bufp)
            ms_start(p, bufp, ssemp)

        @pl.when(i % 2 == 0)
        def _():
            half(0, 1, mbuf0, mbuf1, mgsem0, mgsem1, mssem0, mssem1)

        @pl.when(i % 2 == 1)
        def _():
            half(1, 0, mbuf1, mbuf0, mgsem1, mgsem0, mssem1, mssem0)

        return 0

    lax.fori_loop(0, trips_m, mbody, 0)

    @pl.when(trips_m >= 2)
    def _():
        @pl.when(trips_m % 2 == 0)
        def _():
            ms_wait(0, mbuf0, mssem0)

        @pl.when(trips_m % 2 == 1)
        def _():
            ms_wait(1, mbuf1, mssem1)

    @pl.when(trips_m % 2 == 1)
    def _():
        ms_wait(0, mbuf0, mssem0)

    @pl.when((trips_m >= 1) & (trips_m % 2 == 0))
    def _():
        ms_wait(1, mbuf1, mssem1)


_GR = 16


def _tc_gather_body(idx_ref, *refs):
    m_refs = refs[:_GR]
    o_ref = refs[_GR]
    del idx_ref
    for j in range(_GR):
        o_ref[0, j, :] = m_refs[j][0, 0, :].astype(jnp.float32)


def _tc_gather_masks(idx, masks3):
    grid_spec = pltpu.PrefetchScalarGridSpec(
        num_scalar_prefetch=1,
        grid=(_B // _GR,),
        in_specs=[
            pl.BlockSpec((1, 1, _C),
                         (lambda i, idx_ref, j=j: (idx_ref[_GR * i + j], 0, 0)))
            for j in range(_GR)
        ],
        out_specs=pl.BlockSpec((1, _GR, _C), lambda i, idx_ref: (i, 0, 0)),
    )
    out = pl.pallas_call(
        _tc_gather_body,
        grid_spec=grid_spec,
        out_shape=jax.ShapeDtypeStruct((_B // _GR, _GR, _C), jnp.float32),
        compiler_params=pltpu.CompilerParams(
            dimension_semantics=("arbitrary",),
        ),
    )(idx, *([masks3] * _GR))
    return out.reshape(_B, _C)


def kernel(X, idx, masks):
    n = masks.shape[0]
    xr = X.reshape(_B * _SR, 128)
    bm = _tc_gather_masks(idx.astype(jnp.int32), masks.reshape(n, 1, _C))
    z = jnp.zeros((_DC, 128), jnp.float32)

    mesh = plsc.VectorSubcoreMesh(core_axis_name="c", subcore_axis_name="s")
    run = functools.partial(
        pl.kernel, mesh=mesh,
        out_type=jax.ShapeDtypeStruct((_B * _SR, 128), jnp.float32),
        scratch_types=[
            pltpu.VMEM((_EPW, _C), jnp.float32),
            pltpu.VMEM((_RPW + _KC,), jnp.int32),
            pltpu.VMEM((_RPW + _DC,), jnp.int32),
            pltpu.VMEM((_MXN,), jnp.int32),
            pltpu.VMEM((_MXN,), jnp.int32),
            pltpu.VMEM((_MXN,), jnp.int32),
            pltpu.VMEM((_MXN,), jnp.int32),
            pltpu.VMEM((2, _KC), jnp.int32),
            pltpu.VMEM((4, _DC), jnp.int32),
            pltpu.VMEM((2, _MC), jnp.int32),
            pltpu.VMEM((_KC, 128), jnp.float32),
            pltpu.VMEM((_KC, 128), jnp.float32),
            pltpu.VMEM((_MC, 128), jnp.float32),
            pltpu.VMEM((_MC, 128), jnp.float32),
            pltpu.VMEM((_DC, 128), jnp.float32),
            pltpu.SemaphoreType.DMA,
            pltpu.SemaphoreType.DMA,
            pltpu.SemaphoreType.DMA,
            pltpu.SemaphoreType.DMA,
            pltpu.SemaphoreType.DMA,
            pltpu.SemaphoreType.DMA,
            pltpu.SemaphoreType.DMA,
            pltpu.SemaphoreType.DMA,
            pltpu.SemaphoreType.DMA,
        ],
        compiler_params=pltpu.CompilerParams(needs_layout_passes=False),
    )(_sc_body)
    out = run(xr, bm, z)
    return out.reshape(_B, _C, _H, _W)


# speedup vs baseline: 17.8371x; 8.2370x over previous
import jax
import jax.numpy as jnp
from jax.experimental import pallas as pl
from jax.experimental.pallas import tpu as pltpu

_B, _C, _H, _W = 1024, 256, 14, 14
_HW = _H * _W
_GR = 16
_SLAB = 4


def _tc_gather_body(idx_ref, *refs):
    m_refs = refs[:_GR]
    o_ref = refs[_GR]
    del idx_ref
    for j in range(_GR):
        o_ref[0, j, :] = m_refs[j][0, 0, :].astype(jnp.float32)


def _tc_gather_masks(idx, masks3):
    grid_spec = pltpu.PrefetchScalarGridSpec(
        num_scalar_prefetch=1,
        grid=(_B // _GR,),
        in_specs=[
            pl.BlockSpec((1, 1, _C),
                         (lambda i, idx_ref, j=j: (idx_ref[_GR * i + j], 0, 0)))
            for j in range(_GR)
        ],
        out_specs=pl.BlockSpec((1, _GR, _C), lambda i, idx_ref: (i, 0, 0)),
    )
    out = pl.pallas_call(
        _tc_gather_body,
        grid_spec=grid_spec,
        out_shape=jax.ShapeDtypeStruct((_B // _GR, _GR, _C), jnp.float32),
        compiler_params=pltpu.CompilerParams(
            dimension_semantics=("arbitrary",),
        ),
    )(idx, *([masks3] * _GR))
    return out.reshape(_B, _C)


def _mul_body(m_ref, x_ref, o_ref):
    o_ref[...] = x_ref[...] * m_ref[None]


def kernel(X, idx, masks):
    n = masks.shape[0]
    bm = _tc_gather_masks(idx.astype(jnp.int32), masks.reshape(n, 1, _C))
    xp = jnp.transpose(X, (2, 3, 0, 1)).reshape(_HW, _B, _C)
    outp = pl.pallas_call(
        _mul_body,
        grid=(_HW // _SLAB,),
        in_specs=[
            pl.BlockSpec((_B, _C), lambda i: (0, 0)),
            pl.BlockSpec((_SLAB, _B, _C), lambda i: (i, 0, 0)),
        ],
        out_specs=pl.BlockSpec((_SLAB, _B, _C), lambda i: (i, 0, 0)),
        out_shape=jax.ShapeDtypeStruct((_HW, _B, _C), jnp.float32),
        compiler_params=pltpu.CompilerParams(
            dimension_semantics=("arbitrary",),
        ),
    )(bm, xp)
    return jnp.transpose(outp.reshape(_H, _W, _B, _C), (2, 3, 0, 1))


# speedup vs baseline: 44.1584x; 2.4757x over previous
import functools

import jax
import jax.numpy as jnp
from jax import lax
from jax.experimental import pallas as pl
from jax.experimental.pallas import tpu as pltpu
from jax.experimental.pallas import tpu_sc as plsc

_B, _C, _H, _W = 1024, 256, 14, 14
_HW = _H * _W
_NW = 32
_EPW = _B // _NW
_SLAB = 4


def _sc_gather_body(mf_hbm, idx_hbm, out_hbm, idx_v, rows_v, sem):
    cid = lax.axis_index("c")
    sid = lax.axis_index("s")
    wid = sid * 2 + cid
    b0 = wid * _EPW
    pltpu.sync_copy(idx_hbm.at[pl.ds(b0, _EPW)], idx_v)
    pltpu.async_copy(mf_hbm.at[idx_v], rows_v, sem).wait()
    pltpu.sync_copy(rows_v, out_hbm.at[pl.ds(b0, _EPW)])


def _sc_gather_masks(masksf, idx):
    mesh = plsc.VectorSubcoreMesh(core_axis_name="c", subcore_axis_name="s")
    run = functools.partial(
        pl.kernel, mesh=mesh,
        out_type=jax.ShapeDtypeStruct((_B, _C), jnp.float32),
        scratch_types=[
            pltpu.VMEM((_EPW,), jnp.int32),
            pltpu.VMEM((_EPW, _C), jnp.float32),
            pltpu.SemaphoreType.DMA,
        ],
        compiler_params=pltpu.CompilerParams(needs_layout_passes=False),
    )(_sc_gather_body)
    return run(masksf, idx)


def _mul_body(m_ref, x_ref, o_ref):
    o_ref[...] = x_ref[...] * m_ref[None]


def kernel(X, idx, masks):
    bm = _sc_gather_masks(masks.astype(jnp.float32), idx.astype(jnp.int32))
    xp = jnp.transpose(X, (2, 3, 0, 1)).reshape(_HW, _B, _C)
    outp = pl.pallas_call(
        _mul_body,
        grid=(_HW // _SLAB,),
        in_specs=[
            pl.BlockSpec((_B, _C), lambda i: (0, 0)),
            pl.BlockSpec((_SLAB, _B, _C), lambda i: (i, 0, 0)),
        ],
        out_specs=pl.BlockSpec((_SLAB, _B, _C), lambda i: (i, 0, 0)),
        out_shape=jax.ShapeDtypeStruct((_HW, _B, _C), jnp.float32),
        compiler_params=pltpu.CompilerParams(
            dimension_semantics=("arbitrary",),
        ),
    )(bm, xp)
    return jnp.transpose(outp.reshape(_H, _W, _B, _C), (2, 3, 0, 1))
